# Initial kernel scaffold; baseline (speedup 1.0000x reference)
import jax
import jax.numpy as jnp
from jax.experimental import pallas as pl


def kernel(inputs, edge_index, W1, b1, W2, b2, W3, b3, fcW, fcb):
    raise NotImplementedError("write your pallas kernel here")


# speedup vs baseline: 6.5939x; 6.5939x over previous
import functools

import jax
import jax.numpy as jnp
from jax import lax
from jax.experimental import pallas as pl
from jax.experimental.pallas import tpu as pltpu
from jax.experimental.pallas import tpu_sc as plsc

N = 10000
E = 320000
D_IN = 128
D_H = 32
D_OUT = 16
D_FC = 8

NC = 2
NS = 16
NW = NC * NS
K = 128
NROWS = 10240
DUMMY = N
NROWS2 = 2 * NROWS
CH1 = -(-E // (NW * K))
CH2 = -(-(2 * E) // (NW * K))


def _sc_segment_sum(D, nrows, ch, gather):
  R = nrows // NS
  mesh = plsc.VectorSubcoreMesh(core_axis_name="c", subcore_axis_name="s")
  scratch = [
      pltpu.VMEM_SHARED((nrows, D), jnp.float32),
      pltpu.VMEM((R, D), jnp.float32),
      pltpu.VMEM((K, D), jnp.float32),
      pltpu.VMEM((K,), jnp.int32),
  ]
  if gather:
    scratch.append(pltpu.VMEM((K,), jnp.int32))
  scratch.append(pltpu.SemaphoreType.DMA)

  def body(*refs):
    if gather:
      h_hbm, srcr, dstr, out_hbm, acc, zbuf, rows, idxd, idxs, sem = refs
    else:
      dstr, out_hbm, acc, zbuf, rows, idxd, sem = refs
    c = lax.axis_index("c")
    s = lax.axis_index("s")
    wid = c * NS + s
    zero16 = jnp.zeros((16,), jnp.float32)

    def zrow(i, carry):
      for d in range(D // 16):
        zbuf[i, pl.ds(d * 16, 16)] = zero16
      return carry

    lax.fori_loop(0, R, zrow, 0)
    pltpu.sync_copy(zbuf, acc.at[pl.ds(s * R, R)])
    if not gather:
      one16 = jnp.ones((16,), jnp.float32)

      def orow(i, carry):
        for d in range(D // 16):
          rows[i, pl.ds(d * 16, 16)] = one16
        return carry

      lax.fori_loop(0, K, orow, 0)
    plsc.subcore_barrier()

    def chunk(j, carry):
      pltpu.sync_copy(dstr.at[wid, j], idxd)
      if gather:
        pltpu.sync_copy(srcr.at[wid, j], idxs)
        pltpu.async_copy(h_hbm.at[idxs], rows, sem).wait()
      pltpu.sync_copy(rows, acc.at[idxd], add=True)
      return carry

    lax.fori_loop(0, ch, chunk, 0)
    plsc.subcore_barrier()
    pltpu.sync_copy(acc.at[pl.ds(s * R, R)], out_hbm.at[c, pl.ds(s * R, R)])

  return pl.kernel(
      body,
      out_type=jax.ShapeDtypeStruct((NC, nrows, D), jnp.float32),
      mesh=mesh,
      scratch_types=scratch,
      compiler_params=pltpu.CompilerParams(use_tc_tiling_on_sc=False),
  )


def _tc_first(x, W1, degp):
  def body(x_ref, w_ref, degp_ref, h_ref, s2_ref):
    deg = degp_ref[0] + degp_ref[1]
    degc = jnp.maximum(deg[:, 0:1], 1.0)
    sc = lax.rsqrt(degc)
    s_out = sc[:NROWS]
    s_in = sc[NROWS:]
    s2_ref[...] = jnp.concatenate([s_out, s_in], axis=1)
    h = jnp.dot(x_ref[...], w_ref[...], preferred_element_type=jnp.float32)
    h_ref[...] = h * s_out

  return pl.pallas_call(
      body,
      out_shape=[
          jax.ShapeDtypeStruct((NROWS, D_H), jnp.float32),
          jax.ShapeDtypeStruct((NROWS, 2), jnp.float32),
      ],
  )(x, W1, degp)


def _tc_mid(p, s2, b, W):
  dout = W.shape[1]

  def body(p_ref, s2_ref, b_ref, w_ref, o_ref):
    agg = p_ref[0] + p_ref[1]
    h = jnp.maximum(agg * s2_ref[:, 1:2] + b_ref[...], 0.0)
    hw = jnp.dot(h, w_ref[...], preferred_element_type=jnp.float32)
    o_ref[...] = hw * s2_ref[:, 0:1]

  return pl.pallas_call(
      body, out_shape=jax.ShapeDtypeStruct((NROWS, dout), jnp.float32)
  )(p, s2, b, W)


def _tc_head(p, s2, b3, fcW, fcb):
  def body(p_ref, s2_ref, b_ref, fw_ref, fb_ref, o_ref):
    agg = p_ref[0] + p_ref[1]
    h = jnp.maximum(agg * s2_ref[:, 1:2] + b_ref[...], 0.0)
    rid = lax.broadcasted_iota(jnp.int32, (NROWS, D_OUT), 0)
    h = jnp.where(rid < N, h, 0.0)
    g = jnp.max(h, axis=0, keepdims=True)
    logits = jnp.dot(g, fw_ref[...], preferred_element_type=jnp.float32)
    logits = logits + fb_ref[...]
    m = jnp.max(logits, axis=-1, keepdims=True)
    e = jnp.exp(logits - m)
    o_ref[...] = e / jnp.sum(e, axis=-1, keepdims=True)

  return pl.pallas_call(
      body, out_shape=jax.ShapeDtypeStruct((1, D_FC), jnp.float32)
  )(p, s2, b3, fcW, fcb)


def kernel(inputs, edge_index, W1, b1, W2, b2, W3, b3, fcW, fcb):
  src = edge_index[0]
  dst = edge_index[1]
  x = jnp.pad(inputs, ((0, NROWS - N), (0, 0)))

  ep1 = NW * CH1 * K
  fill1 = jnp.full((ep1 - E,), DUMMY, jnp.int32)
  srcp = jnp.concatenate([src, fill1]).reshape(NW, CH1, K)
  dstp = jnp.concatenate([dst, fill1]).reshape(NW, CH1, K)

  ep2 = NW * CH2 * K
  fill2 = jnp.full((ep2 - 2 * E,), DUMMY, jnp.int32)
  dcomb = jnp.concatenate([src, dst + NROWS, fill2]).reshape(NW, CH2, K)

  degp = _sc_segment_sum(16, NROWS2, CH2, gather=False)(dcomb)
  h1s, s2 = _tc_first(x, W1, degp)
  p1 = _sc_segment_sum(D_H, NROWS, CH1, gather=True)(h1s, srcp, dstp)
  h2s = _tc_mid(p1, s2, b1.reshape(1, -1), W2)
  p2 = _sc_segment_sum(D_H, NROWS, CH1, gather=True)(h2s, srcp, dstp)
  h3s = _tc_mid(p2, s2, b2.reshape(1, -1), W3)
  p3 = _sc_segment_sum(D_OUT, NROWS, CH1, gather=True)(h3s, srcp, dstp)
  return _tc_head(p3, s2, b3.reshape(1, -1), fcW, fcb)


# speedup vs baseline: 13.7835x; 2.0904x over previous
import functools

import jax
import jax.numpy as jnp
from jax import lax
from jax.experimental import pallas as pl
from jax.experimental.pallas import tpu as pltpu
from jax.experimental.pallas import tpu_sc as plsc

N = 10000
E = 320000
D_IN = 128
D_H = 32
D_OUT = 16
D_FC = 8

NC = 2
NS = 16
NW = NC * NS
K = 128
NROWS = 10240
DUMMY = N
NROWS2 = 2 * NROWS
CH1 = -(-E // (NW * K))
CH2 = -(-(2 * E) // (NW * K))


NB = 3


def _sc_segment_sum(D, nrows, ch, gather):
  R = nrows // NS
  mesh = plsc.VectorSubcoreMesh(core_axis_name="c", subcore_axis_name="s")
  scratch = [
      pltpu.VMEM_SHARED((nrows, D), jnp.float32),
      pltpu.VMEM((R, D), jnp.float32),
      pltpu.VMEM((ch, K), jnp.int32),
  ]
  if gather:
    scratch += [
        pltpu.VMEM((ch, K), jnp.int32),
        pltpu.VMEM((NB, K, D), jnp.float32),
    ]
    scratch += [pltpu.SemaphoreType.DMA] * (2 * NB)
  else:
    scratch += [
        pltpu.VMEM((K, D), jnp.float32),
        pltpu.SemaphoreType.DMA,
    ]

  def body(*refs):
    if gather:
      h_hbm, srcr, dstr, out_hbm, acc, zbuf, idxd, idxs, rows, *sems = refs
      gsem = sems[:NB]
      ssem = sems[NB:]
    else:
      dstr, out_hbm, acc, zbuf, idxd, ones, ssem = refs
    c = lax.axis_index("c")
    s = lax.axis_index("s")
    wid = c * NS + s
    pltpu.sync_copy(dstr.at[wid], idxd)
    if gather:
      pltpu.sync_copy(srcr.at[wid], idxs)
      for b in range(NB):
        pltpu.async_copy(h_hbm.at[idxs.at[b]], rows.at[b], gsem[b])
    zero16 = jnp.zeros((16,), jnp.float32)

    def zrow(i, carry):
      for d in range(D // 16):
        zbuf[i, pl.ds(d * 16, 16)] = zero16
      return carry

    lax.fori_loop(0, R, zrow, 0)
    pltpu.sync_copy(zbuf, acc.at[pl.ds(s * R, R)])
    if not gather:
      one16 = jnp.ones((16,), jnp.float32)

      def orow(i, carry):
        for d in range(D // 16):
          ones[i, pl.ds(d * 16, 16)] = one16
        return carry

      lax.fori_loop(0, K, orow, 0)
    plsc.subcore_barrier()

    if gather:
      def rnd(r, carry):
        base = r * NB
        for b in range(NB):
          j = base + b

          @pl.when(j < ch)
          def _():
            pltpu.make_async_copy(
                h_hbm.at[idxs.at[j]], rows.at[b], gsem[b]).wait()
            pltpu.async_copy(rows.at[b], acc.at[idxd.at[j]], ssem[b],
                             add=True)
        for b in range(NB):
          j = base + b

          @pl.when(j < ch)
          def _():
            pltpu.make_async_copy(
                rows.at[b], acc.at[idxd.at[j]], ssem[b]).wait()

          @pl.when(j + NB < ch)
          def _():
            pltpu.async_copy(h_hbm.at[idxs.at[j + NB]], rows.at[b], gsem[b])
        return carry

      lax.fori_loop(0, -(-ch // NB), rnd, 0)
    else:
      def fire(j, carry):
        pltpu.async_copy(ones, acc.at[idxd.at[j]], ssem, add=True)
        return carry

      lax.fori_loop(0, ch, fire, 0)

      def drain(j, carry):
        pltpu.make_async_copy(ones, acc.at[idxd.at[0]], ssem).wait()
        return carry

      lax.fori_loop(0, ch, drain, 0)
    plsc.subcore_barrier()
    pltpu.sync_copy(acc.at[pl.ds(s * R, R)], out_hbm.at[c, pl.ds(s * R, R)])

  return pl.kernel(
      body,
      out_type=jax.ShapeDtypeStruct((NC, nrows, D), jnp.float32),
      mesh=mesh,
      scratch_types=scratch,
      compiler_params=pltpu.CompilerParams(use_tc_tiling_on_sc=False),
  )


def _tc_first(x, W1, degp):
  def body(x_ref, w_ref, degp_ref, h_ref, s2_ref):
    deg = degp_ref[0] + degp_ref[1]
    degc = jnp.maximum(deg[:, 0:1], 1.0)
    sc = lax.rsqrt(degc)
    s_out = sc[:NROWS]
    s_in = sc[NROWS:]
    s2_ref[...] = jnp.concatenate([s_out, s_in], axis=1)
    h = jnp.dot(x_ref[...], w_ref[...], preferred_element_type=jnp.float32)
    h_ref[...] = h * s_out

  return pl.pallas_call(
      body,
      out_shape=[
          jax.ShapeDtypeStruct((NROWS, D_H), jnp.float32),
          jax.ShapeDtypeStruct((NROWS, 2), jnp.float32),
      ],
  )(x, W1, degp)


def _tc_mid(p, s2, b, W):
  dout = W.shape[1]

  def body(p_ref, s2_ref, b_ref, w_ref, o_ref):
    agg = p_ref[0] + p_ref[1]
    h = jnp.maximum(agg * s2_ref[:, 1:2] + b_ref[...], 0.0)
    hw = jnp.dot(h, w_ref[...], preferred_element_type=jnp.float32)
    o_ref[...] = hw * s2_ref[:, 0:1]

  return pl.pallas_call(
      body, out_shape=jax.ShapeDtypeStruct((NROWS, dout), jnp.float32)
  )(p, s2, b, W)


def _tc_head(p, s2, b3, fcW, fcb):
  def body(p_ref, s2_ref, b_ref, fw_ref, fb_ref, o_ref):
    agg = p_ref[0] + p_ref[1]
    h = jnp.maximum(agg * s2_ref[:, 1:2] + b_ref[...], 0.0)
    rid = lax.broadcasted_iota(jnp.int32, (NROWS, D_OUT), 0)
    h = jnp.where(rid < N, h, 0.0)
    g = jnp.max(h, axis=0, keepdims=True)
    logits = jnp.dot(g, fw_ref[...], preferred_element_type=jnp.float32)
    logits = logits + fb_ref[...]
    m = jnp.max(logits, axis=-1, keepdims=True)
    e = jnp.exp(logits - m)
    o_ref[...] = e / jnp.sum(e, axis=-1, keepdims=True)

  return pl.pallas_call(
      body, out_shape=jax.ShapeDtypeStruct((1, D_FC), jnp.float32)
  )(p, s2, b3, fcW, fcb)


def kernel(inputs, edge_index, W1, b1, W2, b2, W3, b3, fcW, fcb):
  src = edge_index[0]
  dst = edge_index[1]
  x = jnp.pad(inputs, ((0, NROWS - N), (0, 0)))

  ep1 = NW * CH1 * K
  fill1 = jnp.full((ep1 - E,), DUMMY, jnp.int32)
  srcp = jnp.concatenate([src, fill1]).reshape(NW, CH1, K)
  dstp = jnp.concatenate([dst, fill1]).reshape(NW, CH1, K)

  ep2 = NW * CH2 * K
  fill2 = jnp.full((ep2 - 2 * E,), DUMMY, jnp.int32)
  dcomb = jnp.concatenate([src, dst + NROWS, fill2]).reshape(NW, CH2, K)

  degp = _sc_segment_sum(16, NROWS2, CH2, gather=False)(dcomb)
  h1s, s2 = _tc_first(x, W1, degp)
  p1 = _sc_segment_sum(D_H, NROWS, CH1, gather=True)(h1s, srcp, dstp)
  h2s = _tc_mid(p1, s2, b1.reshape(1, -1), W2)
  p2 = _sc_segment_sum(D_H, NROWS, CH1, gather=True)(h2s, srcp, dstp)
  h3s = _tc_mid(p2, s2, b2.reshape(1, -1), W3)
  p3 = _sc_segment_sum(D_OUT, NROWS, CH1, gather=True)(h3s, srcp, dstp)
  return _tc_head(p3, s2, b3.reshape(1, -1), fcW, fcb)


# speedup vs baseline: 15.0781x; 1.0939x over previous
import functools

import jax
import jax.numpy as jnp
from jax import lax
from jax.experimental import pallas as pl
from jax.experimental.pallas import tpu as pltpu
from jax.experimental.pallas import tpu_sc as plsc

N = 10000
E = 320000
D_IN = 128
D_H = 32
D_OUT = 16
D_FC = 8

NC = 2
NS = 16
NW = NC * NS
K = 128
NROWS = 10240
DUMMY = N
NROWS2 = 2 * NROWS
CH1 = -(-E // (NW * K))
CH2 = -(-(2 * E) // (NW * K))


NB = 4
ZR = 64
DDEG = 8


def _sc_segment_sum(D, nrows, ch, gather):
  R = nrows // NS
  mesh = plsc.VectorSubcoreMesh(core_axis_name="c", subcore_axis_name="s")
  scratch = [
      pltpu.VMEM_SHARED((nrows, D), jnp.float32),
      pltpu.VMEM((ZR, D), jnp.float32),
      pltpu.VMEM((ch, K), jnp.int32),
  ]
  if gather:
    scratch += [
        pltpu.VMEM((ch, K), jnp.int32),
        pltpu.VMEM((NB, K, D), jnp.float32),
    ]
    scratch += [pltpu.SemaphoreType.DMA] * (2 * NB)
  else:
    scratch += [
        pltpu.VMEM((K, D), jnp.float32),
        pltpu.SemaphoreType.DMA,
    ]

  def body(*refs):
    if gather:
      h_hbm, srcr, dstr, out_hbm, acc, zbuf, idxd, idxs, rows, *sems = refs
      gsem = sems[:NB]
      ssem = sems[NB:]
    else:
      dstr, out_hbm, acc, zbuf, idxd, ones, ssem = refs
    c = lax.axis_index("c")
    s = lax.axis_index("s")
    wid = c * NS + s
    pltpu.sync_copy(dstr.at[wid], idxd)
    if gather:
      pltpu.sync_copy(srcr.at[wid], idxs)
      for b in range(NB):
        pltpu.async_copy(h_hbm.at[idxs.at[b]], rows.at[b], gsem[b])
    zero16 = jnp.zeros((16,), jnp.float32)

    def zrow(i, carry):
      for d in range(D // 16):
        zbuf[i, pl.ds(d * 16, 16)] = zero16
      return carry

    lax.fori_loop(0, ZR, zrow, 0)

    def zcopy(i, carry):
      pltpu.sync_copy(zbuf, acc.at[pl.ds(s * R + i * ZR, ZR)])
      return carry

    lax.fori_loop(0, R // ZR, zcopy, 0)
    if not gather:
      one16 = jnp.ones((16,), jnp.float32)

      def orow(i, carry):
        for d in range(D // 16):
          ones[i, pl.ds(d * 16, 16)] = one16
        return carry

      lax.fori_loop(0, K, orow, 0)
    plsc.subcore_barrier()

    if gather:
      def rnd(r, carry):
        base = r * NB
        for b in range(NB):
          j = base + b

          @pl.when(j < ch)
          def _():
            pltpu.make_async_copy(
                h_hbm.at[idxs.at[j]], rows.at[b], gsem[b]).wait()
            pltpu.async_copy(rows.at[b], acc.at[idxd.at[j]], ssem[b],
                             add=True)
        for b in range(NB):
          j = base + b

          @pl.when(j < ch)
          def _():
            pltpu.make_async_copy(
                rows.at[b], acc.at[idxd.at[j]], ssem[b]).wait()

          @pl.when(j + NB < ch)
          def _():
            pltpu.async_copy(h_hbm.at[idxs.at[j + NB]], rows.at[b], gsem[b])
        return carry

      lax.fori_loop(0, -(-ch // NB), rnd, 0)
    else:
      def fire(j, carry):
        pltpu.async_copy(ones, acc.at[idxd.at[j]], ssem, add=True)
        return carry

      lax.fori_loop(0, ch, fire, 0)

      def drain(j, carry):
        pltpu.make_async_copy(ones, acc.at[idxd.at[0]], ssem).wait()
        return carry

      lax.fori_loop(0, ch, drain, 0)
    plsc.subcore_barrier()
    pltpu.sync_copy(acc.at[pl.ds(s * R, R)], out_hbm.at[c, pl.ds(s * R, R)])

  return pl.kernel(
      body,
      out_type=jax.ShapeDtypeStruct((NC, nrows, D), jnp.float32),
      mesh=mesh,
      scratch_types=scratch,
      compiler_params=pltpu.CompilerParams(use_tc_tiling_on_sc=False),
  )


def _tc_first(x, W1, degp):
  def body(x_ref, w_ref, degp_ref, h_ref, s2_ref):
    deg = degp_ref[0] + degp_ref[1]
    degc = jnp.maximum(deg[:, 0:1], 1.0)
    sc = lax.rsqrt(degc)
    s_out = sc[:NROWS]
    s_in = sc[NROWS:]
    s2_ref[...] = jnp.concatenate([s_out, s_in], axis=1)
    h = jnp.dot(x_ref[...], w_ref[...], preferred_element_type=jnp.float32)
    h_ref[0:N, :] = h * s_out[0:N]
    h_ref[N:NROWS, :] = jnp.zeros((NROWS - N, D_H), jnp.float32)

  return pl.pallas_call(
      body,
      out_shape=[
          jax.ShapeDtypeStruct((NROWS, D_H), jnp.float32),
          jax.ShapeDtypeStruct((NROWS, 2), jnp.float32),
      ],
  )(x, W1, degp)


def _tc_mid(p, s2, b, W):
  dout = W.shape[1]

  def body(p_ref, s2_ref, b_ref, w_ref, o_ref):
    agg = p_ref[0] + p_ref[1]
    h = jnp.maximum(agg * s2_ref[:, 1:2] + b_ref[...], 0.0)
    hw = jnp.dot(h, w_ref[...], preferred_element_type=jnp.float32)
    o_ref[...] = hw * s2_ref[:, 0:1]

  return pl.pallas_call(
      body, out_shape=jax.ShapeDtypeStruct((NROWS, dout), jnp.float32)
  )(p, s2, b, W)


def _tc_head(p, s2, b3, fcW, fcb):
  def body(p_ref, s2_ref, b_ref, fw_ref, fb_ref, o_ref):
    agg = p_ref[0] + p_ref[1]
    h = jnp.maximum(agg * s2_ref[:, 1:2] + b_ref[...], 0.0)
    rid = lax.broadcasted_iota(jnp.int32, (NROWS, D_OUT), 0)
    h = jnp.where(rid < N, h, 0.0)
    g = jnp.max(h, axis=0, keepdims=True)
    logits = jnp.dot(g, fw_ref[...], preferred_element_type=jnp.float32)
    logits = logits + fb_ref[...]
    m = jnp.max(logits, axis=-1, keepdims=True)
    e = jnp.exp(logits - m)
    o_ref[...] = e / jnp.sum(e, axis=-1, keepdims=True)

  return pl.pallas_call(
      body, out_shape=jax.ShapeDtypeStruct((1, D_FC), jnp.float32)
  )(p, s2, b3, fcW, fcb)


def kernel(inputs, edge_index, W1, b1, W2, b2, W3, b3, fcW, fcb):
  src = edge_index[0]
  dst = edge_index[1]

  ep1 = NW * CH1 * K
  fill1 = jnp.full((ep1 - E,), DUMMY, jnp.int32)
  srcp = jnp.concatenate([src, fill1]).reshape(NW, CH1, K)
  dstp = jnp.concatenate([dst, fill1]).reshape(NW, CH1, K)

  ep2 = NW * CH2 * K
  fill2 = jnp.full((ep2 - 2 * E,), DUMMY, jnp.int32)
  dcomb = jnp.concatenate([src, dst + NROWS, fill2]).reshape(NW, CH2, K)

  degp = _sc_segment_sum(DDEG, NROWS2, CH2, gather=False)(dcomb)
  h1s, s2 = _tc_first(inputs, W1, degp)
  p1 = _sc_segment_sum(D_H, NROWS, CH1, gather=True)(h1s, srcp, dstp)
  h2s = _tc_mid(p1, s2, b1.reshape(1, -1), W2)
  p2 = _sc_segment_sum(D_H, NROWS, CH1, gather=True)(h2s, srcp, dstp)
  h3s = _tc_mid(p2, s2, b2.reshape(1, -1), W3)
  p3 = _sc_segment_sum(D_OUT, NROWS, CH1, gather=True)(h3s, srcp, dstp)
  return _tc_head(p3, s2, b3.reshape(1, -1), fcW, fcb)
